# baseline (speedup 1.0000x reference)
import functools

import jax
import jax.numpy as jnp
import numpy as np
from jax import lax
from jax.experimental import pallas as pl
from jax.experimental.pallas import tpu as pltpu
from jax.experimental.pallas import tpu_sc as plsc

N = 10000
E = 320000
D = 128
DP = D // 2
DE = 16

NC = 2
NS = 16
NW = NC * NS
EP = E // NW
B = 96
NFULL = EP // B
REM = EP - NFULL * B
RB = N // NS
ZB = 64
NROWCH = N // ZB
ROWREM = N - NROWCH * ZB

PERM = np.array([32 * (p // 32) + (p % 2) * 16 + (p % 32) // 2
                 for p in range(D)], dtype=np.int32)


def _linear_body(x_ref, w_ref, b_ref, o_ref, *, act):
    y = jnp.dot(x_ref[...], w_ref[...], preferred_element_type=jnp.float32)
    y = y + b_ref[...]
    if act:
        y = y * jax.nn.sigmoid(y)
    o_ref[...] = y.astype(o_ref.dtype)


def _linear(x, w, b, act, block_rows, out_dtype=jnp.float32):
    m, k = x.shape
    dout = w.shape[1]
    return pl.pallas_call(
        functools.partial(_linear_body, act=act),
        grid=(m // block_rows,),
        in_specs=[
            pl.BlockSpec((block_rows, k), lambda i: (i, 0)),
            pl.BlockSpec((k, dout), lambda i: (0, 0)),
            pl.BlockSpec((1, dout), lambda i: (0, 0)),
        ],
        out_specs=pl.BlockSpec((block_rows, dout), lambda i: (i, 0)),
        out_shape=jax.ShapeDtypeStruct((m, dout), out_dtype),
    )(x, w, b.reshape(1, dout))


def _mlp_body(h_ref, p_ref, w1_ref, b1_ref, g1_ref, be1_ref, w2_ref, b2_ref,
              o_ref):
    t = h_ref[...] + p_ref[0] + p_ref[1]
    t = jnp.dot(t, w1_ref[...], preferred_element_type=jnp.float32)
    t = t + b1_ref[...]
    mu = jnp.mean(t, axis=-1, keepdims=True)
    var = jnp.mean((t - mu) ** 2, axis=-1, keepdims=True)
    t = (t - mu) / jnp.sqrt(var + 1e-5) * g1_ref[...] + be1_ref[...]
    t = t * jax.nn.sigmoid(t)
    t = jnp.dot(t, w2_ref[...], preferred_element_type=jnp.float32)
    t = t + b2_ref[...]
    o_ref[...] = t * jax.nn.sigmoid(t)


def _mlp(h, parts, lp, block_rows=2000):
    vec = lambda v: v.reshape(1, D)
    return pl.pallas_call(
        _mlp_body,
        grid=(N // block_rows,),
        in_specs=[
            pl.BlockSpec((block_rows, D), lambda i: (i, 0)),
            pl.BlockSpec((2, block_rows, D), lambda i: (0, i, 0)),
            pl.BlockSpec((D, D), lambda i: (0, 0)),
            pl.BlockSpec((1, D), lambda i: (0, 0)),
            pl.BlockSpec((1, D), lambda i: (0, 0)),
            pl.BlockSpec((1, D), lambda i: (0, 0)),
            pl.BlockSpec((D, D), lambda i: (0, 0)),
            pl.BlockSpec((1, D), lambda i: (0, 0)),
        ],
        out_specs=pl.BlockSpec((block_rows, D), lambda i: (i, 0)),
        out_shape=jax.ShapeDtypeStruct((N, D), jnp.float32),
    )(h, parts, lp['W1'], vec(lp['b1']), vec(lp['g1']), vec(lp['be1']),
      lp['W2'], vec(lp['b2']))


def _unpack_pair(w):
    u = lax.bitcast_convert_type(w, jnp.int32)
    a = lax.bitcast_convert_type(u << 16, jnp.float32)
    bb = lax.bitcast_convert_type(u & jnp.int32(-65536), jnp.float32)
    return a, bb


def _make_agg():
    mesh = plsc.VectorSubcoreMesh(core_axis_name="c", subcore_axis_name="s")

    @functools.partial(
        pl.kernel,
        mesh=mesh,
        out_type=jax.ShapeDtypeStruct((NC, N, D), jnp.float32),
        scratch_types=[
            pltpu.VMEM((B,), jnp.int32),
            pltpu.VMEM((B,), jnp.int32),
            pltpu.VMEM((B,), jnp.int32),
            pltpu.VMEM((B,), jnp.int32),
            pltpu.VMEM((B,), jnp.int32),
            pltpu.VMEM((B,), jnp.int32),
            pltpu.VMEM((B, D), jnp.float32),
            pltpu.VMEM((B, D), jnp.float32),
            pltpu.VMEM((B // 2, D), jnp.float32),
            pltpu.VMEM((B // 2, D), jnp.float32),
            pltpu.VMEM((REM,), jnp.int32),
            pltpu.VMEM((REM,), jnp.int32),
            pltpu.VMEM((REM, D), jnp.float32),
            pltpu.VMEM((REM // 2, D), jnp.float32),
            pltpu.VMEM_SHARED((N, D), jnp.float32),
            pltpu.SemaphoreType.DMA,
            pltpu.SemaphoreType.DMA,
            pltpu.SemaphoreType.DMA,
            pltpu.SemaphoreType.DMA,
            pltpu.SemaphoreType.DMA,
            pltpu.SemaphoreType.DMA,
            pltpu.SemaphoreType.DMA,
            pltpu.SemaphoreType.DMA,
            pltpu.SemaphoreType.DMA,
        ],
    )
    def agg(h_hbm, e_hbm, src_hbm, dst_hbm, out_hbm,
            si0, si1, di0, di1, dsc0, dsc1, rows0, rows1, ev0, ev1,
            si2, di2, rows2, ev2, acc,
            ssi0, ssi1, sde0, sde1, sg0, sg1, ssc0, ssc1, sem2):
        SI = (si0, si1)
        DI = (di0, di1)
        DSC = (dsc0, dsc1)
        ROWS = (rows0, rows1)
        EV = (ev0, ev1)
        SSI = (ssi0, ssi1)
        SDE = (sde0, sde1)
        SG = (sg0, sg1)
        SSC = (ssc0, ssc1)

        cid = lax.axis_index("c")
        sid = lax.axis_index("s")
        wid = sid * NC + cid
        base = wid * EP
        ebase = wid * (EP // 2)

        def prefetch(k, b):
            off = pl.multiple_of(base + k * B, 8)
            eoff = pl.multiple_of(ebase + k * (B // 2), 8)
            pltpu.async_copy(src_hbm.at[pl.ds(off, B)], SI[b], SSI[b])
            pltpu.async_copy(dst_hbm.at[pl.ds(off, B)], DI[b], SDE[b])
            pltpu.async_copy(e_hbm.at[pl.ds(eoff, B // 2)], EV[b], SDE[b])

        def wait_si(b):
            pltpu.make_async_copy(src_hbm.at[pl.ds(0, B)], SI[b], SSI[b]).wait()

        def wait_de(b):
            pltpu.make_async_copy(dst_hbm.at[pl.ds(0, B)], DI[b], SDE[b]).wait()
            pltpu.make_async_copy(e_hbm.at[pl.ds(0, B // 2)], EV[b],
                                  SDE[b]).wait()

        def wait_gather(b):
            pltpu.make_async_copy(h_hbm.at[SI[b]], ROWS[b], SG[b]).wait()

        def wait_scatter(b):
            pltpu.make_async_copy(ROWS[b], acc.at[DSC[b]], SSC[b]).wait()

        prefetch(0, 0)
        prefetch(1, 1)

        zero16 = jnp.zeros((16,), jnp.float32)

        @plsc.parallel_loop(0, ZB)
        def _(i):
            for j in range(D // 16):
                rows0[i, pl.ds(j * 16, 16)] = zero16

        def zchunk(q, _):
            c = sid + q * NS

            @pl.when(c < NROWCH)
            def _():
                pltpu.sync_copy(rows0.at[pl.ds(0, ZB)],
                                acc.at[pl.ds(c * ZB, ZB)])
            return 0
        lax.fori_loop(0, (NROWCH + NS - 1) // NS, zchunk, 0)

        @pl.when(sid == 0)
        def _():
            pltpu.sync_copy(rows0.at[pl.ds(0, ROWREM)],
                            acc.at[pl.ds(NROWCH * ZB, ROWREM)])

        plsc.subcore_barrier()

        wait_si(0)
        pltpu.async_copy(h_hbm.at[SI[0]], ROWS[0], SG[0])

        def chunk_body(k, b, first, gather_guard, pref_guard):
            b1 = 1 - b

            def issue_next():
                wait_si(b1)
                if not first:
                    wait_scatter(b1)
                pltpu.async_copy(h_hbm.at[SI[b1]], ROWS[b1], SG[b1])

            if gather_guard is None:
                issue_next()
            else:
                @pl.when(gather_guard)
                def _():
                    issue_next()

            wait_de(b)
            wait_gather(b)

            @plsc.parallel_loop(0, B // 2, unroll=2)
            def _(i2):
                for eh in range(2):
                    i = i2 * 2 + eh
                    for jw in range(DP // 16):
                        w = EV[b][i2, pl.ds(eh * DP + jw * 16, 16)]
                        ea, eb = _unpack_pair(w)
                        sa = pl.ds(jw * 32, 16)
                        sb = pl.ds(jw * 32 + 16, 16)
                        ROWS[b][i, sa] = jnp.maximum(ROWS[b][i, sa] + ea, 0.0)
                        ROWS[b][i, sb] = jnp.maximum(ROWS[b][i, sb] + eb, 0.0)

            @plsc.parallel_loop(0, B, step=16)
            def _(i):
                DSC[b][pl.ds(i, 16)] = DI[b][pl.ds(i, 16)]

            pltpu.async_copy(ROWS[b], acc.at[DSC[b]], SSC[b], add=True)

            def issue_pref():
                prefetch(k + 2, b)

            if pref_guard is None:
                issue_pref()
            else:
                @pl.when(pref_guard)
                def _():
                    issue_pref()

        chunk_body(0, 0, True, None, None)
        chunk_body(1, 1, False, None, None)

        NPAIR = NFULL // 2

        def pair(q, _):
            k0 = q * 2
            tail = q < NPAIR - 1
            chunk_body(k0, 0, False, None, tail)
            chunk_body(k0 + 1, 1, False, tail, tail)
            return 0
        lax.fori_loop(1, NPAIR, pair, 0)

        wait_scatter(0)
        wait_scatter(1)

        off = base + NFULL * B
        eoff = ebase + NFULL * (B // 2)
        pltpu.sync_copy(src_hbm.at[pl.ds(off, REM)], si2)
        gather = pltpu.async_copy(h_hbm.at[si2], rows2, sem2)
        pltpu.sync_copy(dst_hbm.at[pl.ds(off, REM)], di2)
        pltpu.sync_copy(e_hbm.at[pl.ds(eoff, REM // 2)], ev2)
        gather.wait()

        @plsc.parallel_loop(0, REM // 2)
        def _(i2):
            for eh in range(2):
                i = i2 * 2 + eh
                for jw in range(DP // 16):
                    w = ev2[i2, pl.ds(eh * DP + jw * 16, 16)]
                    ea, eb = _unpack_pair(w)
                    sa = pl.ds(jw * 32, 16)
                    sb = pl.ds(jw * 32 + 16, 16)
                    rows2[i, sa] = jnp.maximum(rows2[i, sa] + ea, 0.0)
                    rows2[i, sb] = jnp.maximum(rows2[i, sb] + eb, 0.0)

        pltpu.sync_copy(rows2, acc.at[di2], add=True)

        plsc.subcore_barrier()

        def ochunk(q, _):
            c = sid + q * NS

            @pl.when(c < NROWCH)
            def _():
                pltpu.sync_copy(acc.at[pl.ds(c * ZB, ZB)],
                                out_hbm.at[cid, pl.ds(c * ZB, ZB)])
            return 0
        lax.fori_loop(0, (NROWCH + NS - 1) // NS, ochunk, 0)

        @pl.when(sid == 0)
        def _():
            pltpu.sync_copy(acc.at[pl.ds(NROWCH * ZB, ROWREM)],
                            out_hbm.at[cid, pl.ds(NROWCH * ZB, ROWREM)])

    return agg


def kernel(x, edge_index, edge_attr, batch, params):
    p = params
    src = edge_index[0]
    dst = edge_index[1]

    h = _linear(x, p['Wa'], p['ba'], act=False, block_rows=2000)
    e16 = _linear(edge_attr, p['Wb'][:, PERM], p['bb'][PERM], act=False,
                  block_rows=4000, out_dtype=jnp.bfloat16)
    ep = lax.bitcast_convert_type(e16.reshape(E // 2, D, 2), jnp.float32)

    agg = _make_agg()
    for lp in p['layers']:
        parts = agg(h, ep, src, dst)
        h = _mlp(h, parts, lp)

    return _linear(h, p['Wl'], p['bl'], act=True, block_rows=2000)


# speedup vs baseline: 17.1864x; 17.1864x over previous
import functools

import jax
import jax.numpy as jnp
import numpy as np
from jax import lax
from jax.experimental import pallas as pl
from jax.experimental.pallas import tpu as pltpu
from jax.experimental.pallas import tpu_sc as plsc

N = 10000
E = 320000
D = 128
DP = D // 2
DE = 16

NC = 2
NS = 16
NW = NC * NS
EP = E // NW
B = 64
NFULL = EP // B
REM = EP - NFULL * B
RB = N // NS
ZB = 64
NROWCH = N // ZB
ROWREM = N - NROWCH * ZB

PA = np.array([32 * (t // 16) + t % 16 for t in range(DP)], dtype=np.int32)
PB = PA + 16


def _linear_body(x_ref, w_ref, b_ref, o_ref, *, act):
    y = jnp.dot(x_ref[...], w_ref[...], preferred_element_type=jnp.float32)
    y = y + b_ref[...]
    if act:
        y = y * jax.nn.sigmoid(y)
    o_ref[...] = y.astype(o_ref.dtype)


def _linear(x, w, b, act, block_rows, out_dtype=jnp.float32):
    m, k = x.shape
    dout = w.shape[1]
    return pl.pallas_call(
        functools.partial(_linear_body, act=act),
        grid=(m // block_rows,),
        in_specs=[
            pl.BlockSpec((block_rows, k), lambda i: (i, 0)),
            pl.BlockSpec((k, dout), lambda i: (0, 0)),
            pl.BlockSpec((1, dout), lambda i: (0, 0)),
        ],
        out_specs=pl.BlockSpec((block_rows, dout), lambda i: (i, 0)),
        out_shape=jax.ShapeDtypeStruct((m, dout), out_dtype),
    )(x, w, b.reshape(1, dout))


def _bf16_bits(y):
    i = lax.bitcast_convert_type(y, jnp.int32)
    return (i + jnp.int32(0x7FFF) + ((i >> 16) & 1)) >> 16


def _epack_body(x_ref, wa_ref, ba_ref, wb_ref, bb_ref, o_ref):
    x = x_ref[...]
    ya = jnp.dot(x, wa_ref[...], preferred_element_type=jnp.float32)
    ya = ya + ba_ref[...]
    yb = jnp.dot(x, wb_ref[...], preferred_element_type=jnp.float32)
    yb = yb + bb_ref[...]
    o_ref[...] = (_bf16_bits(ya) & jnp.int32(0xFFFF)) | (_bf16_bits(yb) << 16)


def _epack(x, w, b, block_rows):
    m, k = x.shape
    return pl.pallas_call(
        _epack_body,
        grid=(m // block_rows,),
        in_specs=[
            pl.BlockSpec((block_rows, k), lambda i: (i, 0)),
            pl.BlockSpec((k, DP), lambda i: (0, 0)),
            pl.BlockSpec((1, DP), lambda i: (0, 0)),
            pl.BlockSpec((k, DP), lambda i: (0, 0)),
            pl.BlockSpec((1, DP), lambda i: (0, 0)),
        ],
        out_specs=pl.BlockSpec((block_rows, DP), lambda i: (i, 0)),
        out_shape=jax.ShapeDtypeStruct((m, DP), jnp.int32),
    )(x, w[:, PA], b[PA].reshape(1, DP), w[:, PB], b[PB].reshape(1, DP))


def _mlp_body(h_ref, p_ref, w1_ref, b1_ref, g1_ref, be1_ref, w2_ref, b2_ref,
              o_ref):
    t = h_ref[...] + p_ref[0] + p_ref[1]
    t = jnp.dot(t, w1_ref[...], preferred_element_type=jnp.float32)
    t = t + b1_ref[...]
    mu = jnp.mean(t, axis=-1, keepdims=True)
    var = jnp.mean((t - mu) ** 2, axis=-1, keepdims=True)
    t = (t - mu) / jnp.sqrt(var + 1e-5) * g1_ref[...] + be1_ref[...]
    t = t * jax.nn.sigmoid(t)
    t = jnp.dot(t, w2_ref[...], preferred_element_type=jnp.float32)
    t = t + b2_ref[...]
    o_ref[...] = t * jax.nn.sigmoid(t)


def _mlp(h, parts, lp, block_rows=2000):
    vec = lambda v: v.reshape(1, D)
    return pl.pallas_call(
        _mlp_body,
        grid=(N // block_rows,),
        in_specs=[
            pl.BlockSpec((block_rows, D), lambda i: (i, 0)),
            pl.BlockSpec((2, block_rows, D), lambda i: (0, i, 0)),
            pl.BlockSpec((D, D), lambda i: (0, 0)),
            pl.BlockSpec((1, D), lambda i: (0, 0)),
            pl.BlockSpec((1, D), lambda i: (0, 0)),
            pl.BlockSpec((1, D), lambda i: (0, 0)),
            pl.BlockSpec((D, D), lambda i: (0, 0)),
            pl.BlockSpec((1, D), lambda i: (0, 0)),
        ],
        out_specs=pl.BlockSpec((block_rows, D), lambda i: (i, 0)),
        out_shape=jax.ShapeDtypeStruct((N, D), jnp.float32),
    )(h, parts, lp['W1'], vec(lp['b1']), vec(lp['g1']), vec(lp['be1']),
      lp['W2'], vec(lp['b2']))


def _unpack_pair(u):
    a = lax.bitcast_convert_type(u << 16, jnp.float32)
    bb = lax.bitcast_convert_type(u & jnp.int32(-65536), jnp.float32)
    return a, bb


def _make_agg():
    mesh = plsc.VectorSubcoreMesh(core_axis_name="c", subcore_axis_name="s")

    @functools.partial(
        pl.kernel,
        mesh=mesh,
        out_type=jax.ShapeDtypeStruct((NC, N, D), jnp.float32),
        scratch_types=[
            pltpu.VMEM((B,), jnp.int32),
            pltpu.VMEM((B,), jnp.int32),
            pltpu.VMEM((B,), jnp.int32),
            pltpu.VMEM((B,), jnp.int32),
            pltpu.VMEM((B,), jnp.int32),
            pltpu.VMEM((B,), jnp.int32),
            pltpu.VMEM((B, D), jnp.float32),
            pltpu.VMEM((B, D), jnp.float32),
            pltpu.VMEM((B, DP), jnp.int32),
            pltpu.VMEM((B, DP), jnp.int32),
            pltpu.VMEM((REM,), jnp.int32),
            pltpu.VMEM((REM,), jnp.int32),
            pltpu.VMEM((REM, D), jnp.float32),
            pltpu.VMEM((REM, DP), jnp.int32),
            pltpu.VMEM_SHARED((N, D), jnp.float32),
            pltpu.SemaphoreType.DMA,
            pltpu.SemaphoreType.DMA,
            pltpu.SemaphoreType.DMA,
            pltpu.SemaphoreType.DMA,
            pltpu.SemaphoreType.DMA,
            pltpu.SemaphoreType.DMA,
            pltpu.SemaphoreType.DMA,
            pltpu.SemaphoreType.DMA,
            pltpu.SemaphoreType.DMA,
        ],
    )
    def agg(h_hbm, e_hbm, src_hbm, dst_hbm, out_hbm,
            si0, si1, di0, di1, dsc0, dsc1, rows0, rows1, ev0, ev1,
            si2, di2, rows2, ev2, acc,
            ssi0, ssi1, sde0, sde1, sg0, sg1, ssc0, ssc1, sem2):
        SI = (si0, si1)
        DI = (di0, di1)
        DSC = (dsc0, dsc1)
        ROWS = (rows0, rows1)
        EV = (ev0, ev1)
        SSI = (ssi0, ssi1)
        SDE = (sde0, sde1)
        SG = (sg0, sg1)
        SSC = (ssc0, ssc1)

        cid = lax.axis_index("c")
        sid = lax.axis_index("s")
        wid = sid * NC + cid
        base = wid * EP

        def prefetch(k, b):
            off = pl.multiple_of(base + k * B, 8)
            pltpu.async_copy(src_hbm.at[pl.ds(off, B)], SI[b], SSI[b])
            pltpu.async_copy(dst_hbm.at[pl.ds(off, B)], DI[b], SDE[b])
            pltpu.async_copy(e_hbm.at[pl.ds(off, B)], EV[b], SDE[b])

        def wait_si(b):
            pltpu.make_async_copy(src_hbm.at[pl.ds(0, B)], SI[b], SSI[b]).wait()

        def wait_de(b):
            pltpu.make_async_copy(dst_hbm.at[pl.ds(0, B)], DI[b], SDE[b]).wait()
            pltpu.make_async_copy(e_hbm.at[pl.ds(0, B)], EV[b],
                                  SDE[b]).wait()

        def wait_gather(b):
            pltpu.make_async_copy(h_hbm.at[SI[b]], ROWS[b], SG[b]).wait()

        def wait_scatter(b):
            pltpu.make_async_copy(ROWS[b], acc.at[DSC[b]], SSC[b]).wait()

        prefetch(0, 0)
        prefetch(1, 1)

        zero16 = jnp.zeros((16,), jnp.float32)

        @plsc.parallel_loop(0, ZB)
        def _(i):
            for j in range(D // 16):
                rows0[i, pl.ds(j * 16, 16)] = zero16

        def zchunk(q, _):
            c = sid + q * NS

            @pl.when(c < NROWCH)
            def _():
                pltpu.sync_copy(rows0.at[pl.ds(0, ZB)],
                                acc.at[pl.ds(c * ZB, ZB)])
            return 0
        lax.fori_loop(0, (NROWCH + NS - 1) // NS, zchunk, 0)

        @pl.when(sid == 0)
        def _():
            pltpu.sync_copy(rows0.at[pl.ds(0, ROWREM)],
                            acc.at[pl.ds(NROWCH * ZB, ROWREM)])

        plsc.subcore_barrier()

        wait_si(0)
        pltpu.async_copy(h_hbm.at[SI[0]], ROWS[0], SG[0])

        def chunk_body(k, b, first, gather_guard, pref_guard):
            b1 = 1 - b

            def issue_next():
                wait_si(b1)
                if not first:
                    wait_scatter(b1)
                pltpu.async_copy(h_hbm.at[SI[b1]], ROWS[b1], SG[b1])

            if gather_guard is None:
                issue_next()
            else:
                @pl.when(gather_guard)
                def _():
                    issue_next()

            wait_de(b)
            wait_gather(b)

            @plsc.parallel_loop(0, B, unroll=2)
            def _(i):
                for jw in range(DP // 16):
                    ea, eb = _unpack_pair(EV[b][i, pl.ds(jw * 16, 16)])
                    sa = pl.ds(jw * 32, 16)
                    sb = pl.ds(jw * 32 + 16, 16)
                    ROWS[b][i, sa] = jnp.maximum(ROWS[b][i, sa] + ea, 0.0)
                    ROWS[b][i, sb] = jnp.maximum(ROWS[b][i, sb] + eb, 0.0)

            @plsc.parallel_loop(0, B, step=16)
            def _(i):
                DSC[b][pl.ds(i, 16)] = DI[b][pl.ds(i, 16)]

            pltpu.async_copy(ROWS[b], acc.at[DSC[b]], SSC[b], add=True)

            def issue_pref():
                prefetch(k + 2, b)

            if pref_guard is None:
                issue_pref()
            else:
                @pl.when(pref_guard)
                def _():
                    issue_pref()

        chunk_body(0, 0, True, None, None)
        chunk_body(1, 1, False, None, None)

        NPAIR = NFULL // 2

        def pair(q, _):
            k0 = q * 2
            tail = q < NPAIR - 1
            chunk_body(k0, 0, False, None, tail)
            chunk_body(k0 + 1, 1, False, tail, tail)
            return 0
        lax.fori_loop(1, NPAIR, pair, 0)

        wait_scatter(0)
        wait_scatter(1)

        off = base + NFULL * B
        pltpu.sync_copy(src_hbm.at[pl.ds(off, REM)], si2)
        gather = pltpu.async_copy(h_hbm.at[si2], rows2, sem2)
        pltpu.sync_copy(dst_hbm.at[pl.ds(off, REM)], di2)
        pltpu.sync_copy(e_hbm.at[pl.ds(off, REM)], ev2)
        gather.wait()

        @plsc.parallel_loop(0, REM)
        def _(i):
            for jw in range(DP // 16):
                ea, eb = _unpack_pair(ev2[i, pl.ds(jw * 16, 16)])
                sa = pl.ds(jw * 32, 16)
                sb = pl.ds(jw * 32 + 16, 16)
                rows2[i, sa] = jnp.maximum(rows2[i, sa] + ea, 0.0)
                rows2[i, sb] = jnp.maximum(rows2[i, sb] + eb, 0.0)

        pltpu.sync_copy(rows2, acc.at[di2], add=True)

        plsc.subcore_barrier()

        def ochunk(q, _):
            c = sid + q * NS

            @pl.when(c < NROWCH)
            def _():
                pltpu.sync_copy(acc.at[pl.ds(c * ZB, ZB)],
                                out_hbm.at[cid, pl.ds(c * ZB, ZB)])
            return 0
        lax.fori_loop(0, (NROWCH + NS - 1) // NS, ochunk, 0)

        @pl.when(sid == 0)
        def _():
            pltpu.sync_copy(acc.at[pl.ds(NROWCH * ZB, ROWREM)],
                            out_hbm.at[cid, pl.ds(NROWCH * ZB, ROWREM)])

    return agg


def kernel(x, edge_index, edge_attr, batch, params):
    p = params
    src = edge_index[0]
    dst = edge_index[1]

    h = _linear(x, p['Wa'], p['ba'], act=False, block_rows=2000)
    ep = _epack(edge_attr, p['Wb'], p['bb'], block_rows=4000)

    agg = _make_agg()
    for lp in p['layers']:
        parts = agg(h, ep, src, dst)
        h = _mlp(h, parts, lp)

    return _linear(h, p['Wl'], p['bl'], act=True, block_rows=2000)


# speedup vs baseline: 17.8077x; 1.0362x over previous
import functools

import jax
import jax.numpy as jnp
import numpy as np
from jax import lax
from jax.experimental import pallas as pl
from jax.experimental.pallas import tpu as pltpu
from jax.experimental.pallas import tpu_sc as plsc

N = 10000
E = 320000
D = 128
DP = D // 2
DE = 16

NC = 2
NS = 16
NW = NC * NS
EP = E // NW
B = 72
NFULL = EP // B
REM = EP - NFULL * B
RB = N // NS
ZB = 64
NROWCH = N // ZB
ROWREM = N - NROWCH * ZB

PA = np.array([32 * (t // 16) + t % 16 for t in range(DP)], dtype=np.int32)
PB = PA + 16


def _linear_body(x_ref, w_ref, b_ref, o_ref, *, act):
    y = jnp.dot(x_ref[...], w_ref[...], preferred_element_type=jnp.float32)
    y = y + b_ref[...]
    if act:
        y = y * jax.nn.sigmoid(y)
    o_ref[...] = y.astype(o_ref.dtype)


def _linear(x, w, b, act, block_rows, out_dtype=jnp.float32):
    m, k = x.shape
    dout = w.shape[1]
    return pl.pallas_call(
        functools.partial(_linear_body, act=act),
        grid=(m // block_rows,),
        in_specs=[
            pl.BlockSpec((block_rows, k), lambda i: (i, 0)),
            pl.BlockSpec((k, dout), lambda i: (0, 0)),
            pl.BlockSpec((1, dout), lambda i: (0, 0)),
        ],
        out_specs=pl.BlockSpec((block_rows, dout), lambda i: (i, 0)),
        out_shape=jax.ShapeDtypeStruct((m, dout), out_dtype),
    )(x, w, b.reshape(1, dout))


def _bf16_bits(y):
    i = lax.bitcast_convert_type(y, jnp.int32)
    return (i + jnp.int32(0x7FFF) + ((i >> 16) & 1)) >> 16


def _epack_body(x_ref, wa_ref, ba_ref, wb_ref, bb_ref, o_ref):
    x = x_ref[...]
    ya = jnp.dot(x, wa_ref[...], preferred_element_type=jnp.float32)
    ya = ya + ba_ref[...]
    yb = jnp.dot(x, wb_ref[...], preferred_element_type=jnp.float32)
    yb = yb + bb_ref[...]
    o_ref[...] = (_bf16_bits(ya) & jnp.int32(0xFFFF)) | (_bf16_bits(yb) << 16)


def _epack(x, w, b, block_rows):
    m, k = x.shape
    return pl.pallas_call(
        _epack_body,
        grid=(m // block_rows,),
        in_specs=[
            pl.BlockSpec((block_rows, k), lambda i: (i, 0)),
            pl.BlockSpec((k, DP), lambda i: (0, 0)),
            pl.BlockSpec((1, DP), lambda i: (0, 0)),
            pl.BlockSpec((k, DP), lambda i: (0, 0)),
            pl.BlockSpec((1, DP), lambda i: (0, 0)),
        ],
        out_specs=pl.BlockSpec((block_rows, DP), lambda i: (i, 0)),
        out_shape=jax.ShapeDtypeStruct((m, DP), jnp.int32),
    )(x, w[:, PA], b[PA].reshape(1, DP), w[:, PB], b[PB].reshape(1, DP))


def _mlp_body(h_ref, p_ref, w1_ref, b1_ref, g1_ref, be1_ref, w2_ref, b2_ref,
              o_ref, *, wl_ref=None, bl_ref=None):
    t = h_ref[...] + p_ref[0] + p_ref[1]
    t = jnp.dot(t, w1_ref[...], preferred_element_type=jnp.float32)
    t = t + b1_ref[...]
    mu = jnp.mean(t, axis=-1, keepdims=True)
    var = jnp.mean((t - mu) ** 2, axis=-1, keepdims=True)
    t = (t - mu) / jnp.sqrt(var + 1e-5) * g1_ref[...] + be1_ref[...]
    t = t * jax.nn.sigmoid(t)
    t = jnp.dot(t, w2_ref[...], preferred_element_type=jnp.float32)
    t = t + b2_ref[...]
    t = t * jax.nn.sigmoid(t)
    if wl_ref is not None:
        t = jnp.dot(t, wl_ref[...], preferred_element_type=jnp.float32)
        t = t + bl_ref[...]
        t = t * jax.nn.sigmoid(t)
    o_ref[...] = t


def _mlp_final_body(h_ref, p_ref, w1_ref, b1_ref, g1_ref, be1_ref, w2_ref,
                    b2_ref, wl_ref, bl_ref, o_ref):
    _mlp_body(h_ref, p_ref, w1_ref, b1_ref, g1_ref, be1_ref, w2_ref, b2_ref,
              o_ref, wl_ref=wl_ref, bl_ref=bl_ref)


def _mlp(h, parts, lp, final=None, block_rows=2000):
    vec = lambda v: v.reshape(1, D)
    mat_spec = pl.BlockSpec((D, D), lambda i: (0, 0))
    vec_spec = pl.BlockSpec((1, D), lambda i: (0, 0))
    row_spec = pl.BlockSpec((block_rows, D), lambda i: (i, 0))
    in_specs = [
        row_spec,
        pl.BlockSpec((2, block_rows, D), lambda i: (0, i, 0)),
        mat_spec, vec_spec, vec_spec, vec_spec, mat_spec, vec_spec,
    ]
    args = [h, parts, lp['W1'], vec(lp['b1']), vec(lp['g1']), vec(lp['be1']),
            lp['W2'], vec(lp['b2'])]
    body = _mlp_body
    if final is not None:
        wl, bl = final
        body = _mlp_final_body
        in_specs += [mat_spec, vec_spec]
        args += [wl, vec(bl)]
    return pl.pallas_call(
        body,
        grid=(N // block_rows,),
        in_specs=in_specs,
        out_specs=row_spec,
        out_shape=jax.ShapeDtypeStruct((N, D), jnp.float32),
    )(*args)


def _unpack_pair(u):
    a = lax.bitcast_convert_type(u << 16, jnp.float32)
    bb = lax.bitcast_convert_type(u & jnp.int32(-65536), jnp.float32)
    return a, bb


def _make_agg():
    mesh = plsc.VectorSubcoreMesh(core_axis_name="c", subcore_axis_name="s")

    @functools.partial(
        pl.kernel,
        mesh=mesh,
        out_type=jax.ShapeDtypeStruct((NC, N, D), jnp.float32),
        scratch_types=[
            pltpu.VMEM((B,), jnp.int32),
            pltpu.VMEM((B,), jnp.int32),
            pltpu.VMEM((B,), jnp.int32),
            pltpu.VMEM((B,), jnp.int32),
            pltpu.VMEM((B,), jnp.int32),
            pltpu.VMEM((B,), jnp.int32),
            pltpu.VMEM((B, D), jnp.float32),
            pltpu.VMEM((B, D), jnp.float32),
            pltpu.VMEM((B, DP), jnp.int32),
            pltpu.VMEM((B, DP), jnp.int32),
            pltpu.VMEM((REM,), jnp.int32),
            pltpu.VMEM((REM,), jnp.int32),
            pltpu.VMEM_SHARED((N, D), jnp.float32),
            pltpu.SemaphoreType.DMA,
            pltpu.SemaphoreType.DMA,
            pltpu.SemaphoreType.DMA,
            pltpu.SemaphoreType.DMA,
            pltpu.SemaphoreType.DMA,
            pltpu.SemaphoreType.DMA,
            pltpu.SemaphoreType.DMA,
            pltpu.SemaphoreType.DMA,
            pltpu.SemaphoreType.DMA,
        ],
    )
    def agg(h_hbm, e_hbm, src_hbm, dst_hbm, out_hbm,
            si0, si1, di0, di1, dsc0, dsc1, rows0, rows1, ev0, ev1,
            si2, di2, acc,
            ssi0, ssi1, sde0, sde1, sg0, sg1, ssc0, ssc1, sem2):
        SI = (si0, si1)
        DI = (di0, di1)
        DSC = (dsc0, dsc1)
        ROWS = (rows0, rows1)
        EV = (ev0, ev1)
        SSI = (ssi0, ssi1)
        SDE = (sde0, sde1)
        SG = (sg0, sg1)
        SSC = (ssc0, ssc1)

        cid = lax.axis_index("c")
        sid = lax.axis_index("s")
        wid = sid * NC + cid
        base = wid * EP

        def prefetch(k, b):
            off = pl.multiple_of(base + k * B, 8)
            pltpu.async_copy(src_hbm.at[pl.ds(off, B)], SI[b], SSI[b])
            pltpu.async_copy(dst_hbm.at[pl.ds(off, B)], DI[b], SDE[b])
            pltpu.async_copy(e_hbm.at[pl.ds(off, B)], EV[b], SDE[b])

        def wait_si(b):
            pltpu.make_async_copy(src_hbm.at[pl.ds(0, B)], SI[b], SSI[b]).wait()

        def wait_de(b):
            pltpu.make_async_copy(dst_hbm.at[pl.ds(0, B)], DI[b], SDE[b]).wait()
            pltpu.make_async_copy(e_hbm.at[pl.ds(0, B)], EV[b],
                                  SDE[b]).wait()

        def wait_gather(b):
            pltpu.make_async_copy(h_hbm.at[SI[b]], ROWS[b], SG[b]).wait()

        def wait_scatter(b):
            pltpu.make_async_copy(ROWS[b], acc.at[DSC[b]], SSC[b]).wait()

        prefetch(0, 0)
        prefetch(1, 1)

        zero16 = jnp.zeros((16,), jnp.float32)

        @plsc.parallel_loop(0, ZB)
        def _(i):
            for j in range(D // 16):
                rows0[i, pl.ds(j * 16, 16)] = zero16

        def zchunk(q, _):
            c = sid + q * NS

            @pl.when(c < NROWCH)
            def _():
                pltpu.sync_copy(rows0.at[pl.ds(0, ZB)],
                                acc.at[pl.ds(c * ZB, ZB)])
            return 0
        lax.fori_loop(0, (NROWCH + NS - 1) // NS, zchunk, 0)

        @pl.when(sid == 0)
        def _():
            pltpu.sync_copy(rows0.at[pl.ds(0, ROWREM)],
                            acc.at[pl.ds(NROWCH * ZB, ROWREM)])

        plsc.subcore_barrier()

        wait_si(0)
        pltpu.async_copy(h_hbm.at[SI[0]], ROWS[0], SG[0])

        def chunk_body(k, b, first, gather_guard, pref_guard):
            b1 = 1 - b

            def issue_next():
                wait_si(b1)
                if not first:
                    wait_scatter(b1)
                pltpu.async_copy(h_hbm.at[SI[b1]], ROWS[b1], SG[b1])

            if gather_guard is None:
                issue_next()
            else:
                @pl.when(gather_guard)
                def _():
                    issue_next()

            wait_de(b)
            wait_gather(b)

            @plsc.parallel_loop(0, B, unroll=2)
            def _(i):
                for jw in range(DP // 16):
                    ea, eb = _unpack_pair(EV[b][i, pl.ds(jw * 16, 16)])
                    sa = pl.ds(jw * 32, 16)
                    sb = pl.ds(jw * 32 + 16, 16)
                    ROWS[b][i, sa] = jnp.maximum(ROWS[b][i, sa] + ea, 0.0)
                    ROWS[b][i, sb] = jnp.maximum(ROWS[b][i, sb] + eb, 0.0)

            @plsc.parallel_loop(0, B, step=16)
            def _(i):
                DSC[b][pl.ds(i, 16)] = DI[b][pl.ds(i, 16)]

            pltpu.async_copy(ROWS[b], acc.at[DSC[b]], SSC[b], add=True)

            def issue_pref():
                prefetch(k + 2, b)

            if pref_guard is None:
                issue_pref()
            else:
                @pl.when(pref_guard)
                def _():
                    issue_pref()

        chunk_body(0, 0, True, None, None)
        chunk_body(1, 1, False, None, None)

        NPAIR = NFULL // 2

        def pair(q, _):
            k0 = q * 2
            tail = q < NPAIR - 1
            chunk_body(k0, 0, False, None, tail)
            chunk_body(k0 + 1, 1, False, tail, tail)
            return 0
        lax.fori_loop(1, NPAIR, pair, 0)

        wait_scatter(0)
        wait_scatter(1)

        off = base + NFULL * B
        pltpu.sync_copy(src_hbm.at[pl.ds(off, REM)], si2)
        gather = pltpu.async_copy(h_hbm.at[si2], rows0.at[pl.ds(0, REM)],
                                  sem2)
        pltpu.sync_copy(dst_hbm.at[pl.ds(off, REM)], di2)
        pltpu.sync_copy(e_hbm.at[pl.ds(off, REM)], ev0.at[pl.ds(0, REM)])
        gather.wait()

        @plsc.parallel_loop(0, REM)
        def _(i):
            for jw in range(DP // 16):
                ea, eb = _unpack_pair(ev0[i, pl.ds(jw * 16, 16)])
                sa = pl.ds(jw * 32, 16)
                sb = pl.ds(jw * 32 + 16, 16)
                rows0[i, sa] = jnp.maximum(rows0[i, sa] + ea, 0.0)
                rows0[i, sb] = jnp.maximum(rows0[i, sb] + eb, 0.0)

        pltpu.sync_copy(rows0.at[pl.ds(0, REM)], acc.at[di2], add=True)

        plsc.subcore_barrier()

        def ochunk(q, _):
            c = sid + q * NS

            @pl.when(c < NROWCH)
            def _():
                pltpu.sync_copy(acc.at[pl.ds(c * ZB, ZB)],
                                out_hbm.at[cid, pl.ds(c * ZB, ZB)])
            return 0
        lax.fori_loop(0, (NROWCH + NS - 1) // NS, ochunk, 0)

        @pl.when(sid == 0)
        def _():
            pltpu.sync_copy(acc.at[pl.ds(NROWCH * ZB, ROWREM)],
                            out_hbm.at[cid, pl.ds(NROWCH * ZB, ROWREM)])

    return agg


def kernel(x, edge_index, edge_attr, batch, params):
    p = params
    src = edge_index[0]
    dst = edge_index[1]

    h = _linear(x, p['Wa'], p['ba'], act=False, block_rows=2000)
    ep = _epack(edge_attr, p['Wb'], p['bb'], block_rows=4000)

    agg = _make_agg()
    nl = len(p['layers'])
    for i, lp in enumerate(p['layers']):
        parts = agg(h, ep, src, dst)
        final = (p['Wl'], p['bl']) if i + 1 == nl else None
        h = _mlp(h, parts, lp, final=final)

    return h


# speedup vs baseline: 17.8774x; 1.0039x over previous
import functools

import jax
import jax.numpy as jnp
import numpy as np
from jax import lax
from jax.experimental import pallas as pl
from jax.experimental.pallas import tpu as pltpu
from jax.experimental.pallas import tpu_sc as plsc

N = 10000
E = 320000
D = 128
DP = D // 2
DE = 16

NC = 2
NS = 16
NW = NC * NS
EP = E // NW
B = 72
NFULL = EP // B
REM = EP - NFULL * B
RB = N // NS
ZB = 64
NROWCH = N // ZB
ROWREM = N - NROWCH * ZB

PA = np.array([32 * (t // 16) + t % 16 for t in range(DP)], dtype=np.int32)
PB = PA + 16


def _linear_body(x_ref, w_ref, b_ref, o_ref, *, act):
    y = jnp.dot(x_ref[...], w_ref[...], preferred_element_type=jnp.float32)
    y = y + b_ref[...]
    if act:
        y = y * jax.nn.sigmoid(y)
    o_ref[...] = y.astype(o_ref.dtype)


def _linear(x, w, b, act, block_rows, out_dtype=jnp.float32):
    m, k = x.shape
    dout = w.shape[1]
    return pl.pallas_call(
        functools.partial(_linear_body, act=act),
        grid=(m // block_rows,),
        in_specs=[
            pl.BlockSpec((block_rows, k), lambda i: (i, 0)),
            pl.BlockSpec((k, dout), lambda i: (0, 0)),
            pl.BlockSpec((1, dout), lambda i: (0, 0)),
        ],
        out_specs=pl.BlockSpec((block_rows, dout), lambda i: (i, 0)),
        out_shape=jax.ShapeDtypeStruct((m, dout), out_dtype),
    )(x, w, b.reshape(1, dout))


def _bf16_bits(y):
    i = lax.bitcast_convert_type(y, jnp.int32)
    return (i + jnp.int32(0x7FFF) + ((i >> 16) & 1)) >> 16


def _epack_body(x_ref, wa_ref, ba_ref, wb_ref, bb_ref, o_ref):
    x = x_ref[...]
    ya = jnp.dot(x, wa_ref[...], preferred_element_type=jnp.float32)
    ya = ya + ba_ref[...]
    yb = jnp.dot(x, wb_ref[...], preferred_element_type=jnp.float32)
    yb = yb + bb_ref[...]
    o_ref[...] = (_bf16_bits(ya) & jnp.int32(0xFFFF)) | (_bf16_bits(yb) << 16)


def _enc_body(ea_ref, wa_ref, ba_ref, wb_ref, bb_ref, x_ref, wx_ref, bx_ref,
              oe_ref, oh_ref, *, nh):
    _epack_body(ea_ref, wa_ref, ba_ref, wb_ref, bb_ref, oe_ref)

    @pl.when(pl.program_id(0) < nh)
    def _():
        y = jnp.dot(x_ref[...], wx_ref[...],
                    preferred_element_type=jnp.float32)
        oh_ref[...] = y + bx_ref[...]


def _encoders(edge_attr, wb, bb, x, wa, ba, eblock, hblock):
    m, k = edge_attr.shape
    grid = m // eblock
    nh = N // hblock
    hmap = lambda i: (jnp.minimum(i, nh - 1), 0)
    return pl.pallas_call(
        functools.partial(_enc_body, nh=nh),
        grid=(grid,),
        in_specs=[
            pl.BlockSpec((eblock, k), lambda i: (i, 0)),
            pl.BlockSpec((k, DP), lambda i: (0, 0)),
            pl.BlockSpec((1, DP), lambda i: (0, 0)),
            pl.BlockSpec((k, DP), lambda i: (0, 0)),
            pl.BlockSpec((1, DP), lambda i: (0, 0)),
            pl.BlockSpec((hblock, D), hmap),
            pl.BlockSpec((D, D), lambda i: (0, 0)),
            pl.BlockSpec((1, D), lambda i: (0, 0)),
        ],
        out_specs=[
            pl.BlockSpec((eblock, DP), lambda i: (i, 0)),
            pl.BlockSpec((hblock, D), hmap),
        ],
        out_shape=[
            jax.ShapeDtypeStruct((m, DP), jnp.int32),
            jax.ShapeDtypeStruct((N, D), jnp.float32),
        ],
    )(edge_attr, wb[:, PA], bb[PA].reshape(1, DP), wb[:, PB],
      bb[PB].reshape(1, DP), x, wa, ba.reshape(1, D))


def _mlp_body(h_ref, p_ref, w1_ref, b1_ref, g1_ref, be1_ref, w2_ref, b2_ref,
              o_ref, *, wl_ref=None, bl_ref=None):
    t = h_ref[...] + p_ref[0] + p_ref[1]
    t = jnp.dot(t, w1_ref[...], preferred_element_type=jnp.float32)
    t = t + b1_ref[...]
    mu = jnp.mean(t, axis=-1, keepdims=True)
    var = jnp.mean((t - mu) ** 2, axis=-1, keepdims=True)
    t = (t - mu) / jnp.sqrt(var + 1e-5) * g1_ref[...] + be1_ref[...]
    t = t * jax.nn.sigmoid(t)
    t = jnp.dot(t, w2_ref[...], preferred_element_type=jnp.float32)
    t = t + b2_ref[...]
    t = t * jax.nn.sigmoid(t)
    if wl_ref is not None:
        t = jnp.dot(t, wl_ref[...], preferred_element_type=jnp.float32)
        t = t + bl_ref[...]
        t = t * jax.nn.sigmoid(t)
    o_ref[...] = t


def _mlp_final_body(h_ref, p_ref, w1_ref, b1_ref, g1_ref, be1_ref, w2_ref,
                    b2_ref, wl_ref, bl_ref, o_ref):
    _mlp_body(h_ref, p_ref, w1_ref, b1_ref, g1_ref, be1_ref, w2_ref, b2_ref,
              o_ref, wl_ref=wl_ref, bl_ref=bl_ref)


def _mlp(h, parts, lp, final=None, block_rows=2000):
    vec = lambda v: v.reshape(1, D)
    mat_spec = pl.BlockSpec((D, D), lambda i: (0, 0))
    vec_spec = pl.BlockSpec((1, D), lambda i: (0, 0))
    row_spec = pl.BlockSpec((block_rows, D), lambda i: (i, 0))
    in_specs = [
        row_spec,
        pl.BlockSpec((2, block_rows, D), lambda i: (0, i, 0)),
        mat_spec, vec_spec, vec_spec, vec_spec, mat_spec, vec_spec,
    ]
    args = [h, parts, lp['W1'], vec(lp['b1']), vec(lp['g1']), vec(lp['be1']),
            lp['W2'], vec(lp['b2'])]
    body = _mlp_body
    if final is not None:
        wl, bl = final
        body = _mlp_final_body
        in_specs += [mat_spec, vec_spec]
        args += [wl, vec(bl)]
    return pl.pallas_call(
        body,
        grid=(N // block_rows,),
        in_specs=in_specs,
        out_specs=row_spec,
        out_shape=jax.ShapeDtypeStruct((N, D), jnp.float32),
    )(*args)


def _unpack_pair(u):
    a = lax.bitcast_convert_type(u << 16, jnp.float32)
    bb = lax.bitcast_convert_type(u & jnp.int32(-65536), jnp.float32)
    return a, bb


def _make_agg():
    mesh = plsc.VectorSubcoreMesh(core_axis_name="c", subcore_axis_name="s")

    @functools.partial(
        pl.kernel,
        mesh=mesh,
        out_type=jax.ShapeDtypeStruct((NC, N, D), jnp.float32),
        scratch_types=[
            pltpu.VMEM((B,), jnp.int32),
            pltpu.VMEM((B,), jnp.int32),
            pltpu.VMEM((B,), jnp.int32),
            pltpu.VMEM((B,), jnp.int32),
            pltpu.VMEM((B,), jnp.int32),
            pltpu.VMEM((B,), jnp.int32),
            pltpu.VMEM((B, D), jnp.float32),
            pltpu.VMEM((B, D), jnp.float32),
            pltpu.VMEM((B, DP), jnp.int32),
            pltpu.VMEM((B, DP), jnp.int32),
            pltpu.VMEM((REM,), jnp.int32),
            pltpu.VMEM((REM,), jnp.int32),
            pltpu.VMEM_SHARED((N, D), jnp.float32),
            pltpu.SemaphoreType.DMA,
            pltpu.SemaphoreType.DMA,
            pltpu.SemaphoreType.DMA,
            pltpu.SemaphoreType.DMA,
            pltpu.SemaphoreType.DMA,
            pltpu.SemaphoreType.DMA,
            pltpu.SemaphoreType.DMA,
            pltpu.SemaphoreType.DMA,
            pltpu.SemaphoreType.DMA,
        ],
    )
    def agg(h_hbm, e_hbm, src_hbm, dst_hbm, out_hbm,
            si0, si1, di0, di1, dsc0, dsc1, rows0, rows1, ev0, ev1,
            si2, di2, acc,
            ssi0, ssi1, sde0, sde1, sg0, sg1, ssc0, ssc1, sem2):
        SI = (si0, si1)
        DI = (di0, di1)
        DSC = (dsc0, dsc1)
        ROWS = (rows0, rows1)
        EV = (ev0, ev1)
        SSI = (ssi0, ssi1)
        SDE = (sde0, sde1)
        SG = (sg0, sg1)
        SSC = (ssc0, ssc1)

        cid = lax.axis_index("c")
        sid = lax.axis_index("s")
        wid = sid * NC + cid
        base = wid * EP

        def prefetch(k, b):
            off = pl.multiple_of(base + k * B, 8)
            pltpu.async_copy(src_hbm.at[pl.ds(off, B)], SI[b], SSI[b])
            pltpu.async_copy(dst_hbm.at[pl.ds(off, B)], DI[b], SDE[b])
            pltpu.async_copy(e_hbm.at[pl.ds(off, B)], EV[b], SDE[b])

        def wait_si(b):
            pltpu.make_async_copy(src_hbm.at[pl.ds(0, B)], SI[b], SSI[b]).wait()

        def wait_de(b):
            pltpu.make_async_copy(dst_hbm.at[pl.ds(0, B)], DI[b], SDE[b]).wait()
            pltpu.make_async_copy(e_hbm.at[pl.ds(0, B)], EV[b],
                                  SDE[b]).wait()

        def wait_gather(b):
            pltpu.make_async_copy(h_hbm.at[SI[b]], ROWS[b], SG[b]).wait()

        def wait_scatter(b):
            pltpu.make_async_copy(ROWS[b], acc.at[DSC[b]], SSC[b]).wait()

        prefetch(0, 0)
        prefetch(1, 1)

        zero16 = jnp.zeros((16,), jnp.float32)

        @plsc.parallel_loop(0, ZB)
        def _(i):
            for j in range(D // 16):
                rows0[i, pl.ds(j * 16, 16)] = zero16

        def zchunk(q, _):
            c = sid + q * NS

            @pl.when(c < NROWCH)
            def _():
                pltpu.sync_copy(rows0.at[pl.ds(0, ZB)],
                                acc.at[pl.ds(c * ZB, ZB)])
            return 0
        lax.fori_loop(0, (NROWCH + NS - 1) // NS, zchunk, 0)

        @pl.when(sid == 0)
        def _():
            pltpu.sync_copy(rows0.at[pl.ds(0, ROWREM)],
                            acc.at[pl.ds(NROWCH * ZB, ROWREM)])

        plsc.subcore_barrier()

        wait_si(0)
        pltpu.async_copy(h_hbm.at[SI[0]], ROWS[0], SG[0])

        def chunk_body(k, b, first, gather_guard, pref_guard):
            b1 = 1 - b

            def issue_next():
                wait_si(b1)
                if not first:
                    wait_scatter(b1)
                pltpu.async_copy(h_hbm.at[SI[b1]], ROWS[b1], SG[b1])

            if gather_guard is None:
                issue_next()
            else:
                @pl.when(gather_guard)
                def _():
                    issue_next()

            wait_de(b)
            wait_gather(b)

            @plsc.parallel_loop(0, B, unroll=2)
            def _(i):
                for jw in range(DP // 16):
                    ea, eb = _unpack_pair(EV[b][i, pl.ds(jw * 16, 16)])
                    sa = pl.ds(jw * 32, 16)
                    sb = pl.ds(jw * 32 + 16, 16)
                    ROWS[b][i, sa] = jnp.maximum(ROWS[b][i, sa] + ea, 0.0)
                    ROWS[b][i, sb] = jnp.maximum(ROWS[b][i, sb] + eb, 0.0)

            @plsc.parallel_loop(0, B, step=16)
            def _(i):
                DSC[b][pl.ds(i, 16)] = DI[b][pl.ds(i, 16)]

            pltpu.async_copy(ROWS[b], acc.at[DSC[b]], SSC[b], add=True)

            def issue_pref():
                prefetch(k + 2, b)

            if pref_guard is None:
                issue_pref()
            else:
                @pl.when(pref_guard)
                def _():
                    issue_pref()

        chunk_body(0, 0, True, None, None)
        chunk_body(1, 1, False, None, None)

        NPAIR = NFULL // 2

        def pair(q, _):
            k0 = q * 2
            tail = q < NPAIR - 1
            chunk_body(k0, 0, False, None, tail)
            chunk_body(k0 + 1, 1, False, tail, tail)
            return 0
        lax.fori_loop(1, NPAIR, pair, 0)

        wait_scatter(0)
        wait_scatter(1)

        off = base + NFULL * B
        pltpu.sync_copy(src_hbm.at[pl.ds(off, REM)], si2)
        gather = pltpu.async_copy(h_hbm.at[si2], rows0.at[pl.ds(0, REM)],
                                  sem2)
        pltpu.sync_copy(dst_hbm.at[pl.ds(off, REM)], di2)
        pltpu.sync_copy(e_hbm.at[pl.ds(off, REM)], ev0.at[pl.ds(0, REM)])
        gather.wait()

        @plsc.parallel_loop(0, REM)
        def _(i):
            for jw in range(DP // 16):
                ea, eb = _unpack_pair(ev0[i, pl.ds(jw * 16, 16)])
                sa = pl.ds(jw * 32, 16)
                sb = pl.ds(jw * 32 + 16, 16)
                rows0[i, sa] = jnp.maximum(rows0[i, sa] + ea, 0.0)
                rows0[i, sb] = jnp.maximum(rows0[i, sb] + eb, 0.0)

        pltpu.sync_copy(rows0.at[pl.ds(0, REM)], acc.at[di2], add=True)

        plsc.subcore_barrier()

        def ochunk(q, _):
            c = sid + q * NS

            @pl.when(c < NROWCH)
            def _():
                pltpu.sync_copy(acc.at[pl.ds(c * ZB, ZB)],
                                out_hbm.at[cid, pl.ds(c * ZB, ZB)])
            return 0
        lax.fori_loop(0, (NROWCH + NS - 1) // NS, ochunk, 0)

        @pl.when(sid == 0)
        def _():
            pltpu.sync_copy(acc.at[pl.ds(NROWCH * ZB, ROWREM)],
                            out_hbm.at[cid, pl.ds(NROWCH * ZB, ROWREM)])

    return agg


def kernel(x, edge_index, edge_attr, batch, params):
    p = params
    src = edge_index[0]
    dst = edge_index[1]

    ep, h = _encoders(edge_attr, p['Wb'], p['bb'], x, p['Wa'], p['ba'],
                      eblock=4000, hblock=2000)

    agg = _make_agg()
    nl = len(p['layers'])
    for i, lp in enumerate(p['layers']):
        parts = agg(h, ep, src, dst)
        final = (p['Wl'], p['bl']) if i + 1 == nl else None
        h = _mlp(h, parts, lp, final=final)

    return h


# speedup vs baseline: 17.8967x; 1.0011x over previous
import functools

import jax
import jax.numpy as jnp
import numpy as np
from jax import lax
from jax.experimental import pallas as pl
from jax.experimental.pallas import tpu as pltpu
from jax.experimental.pallas import tpu_sc as plsc

N = 10000
E = 320000
D = 128
DP = D // 2
DE = 16

NC = 2
NS = 16
NW = NC * NS
EP = E // NW
B = 72
NFULL = EP // B
REM = EP - NFULL * B
RB = N // NS
ZB = 64
NROWCH = N // ZB
ROWREM = N - NROWCH * ZB

PA = np.array([32 * (t // 16) + t % 16 for t in range(DP)], dtype=np.int32)
PB = PA + 16


def _linear_body(x_ref, w_ref, b_ref, o_ref, *, act):
    y = jnp.dot(x_ref[...], w_ref[...], preferred_element_type=jnp.float32)
    y = y + b_ref[...]
    if act:
        y = y * jax.nn.sigmoid(y)
    o_ref[...] = y.astype(o_ref.dtype)


def _linear(x, w, b, act, block_rows, out_dtype=jnp.float32):
    m, k = x.shape
    dout = w.shape[1]
    return pl.pallas_call(
        functools.partial(_linear_body, act=act),
        grid=(m // block_rows,),
        in_specs=[
            pl.BlockSpec((block_rows, k), lambda i: (i, 0)),
            pl.BlockSpec((k, dout), lambda i: (0, 0)),
            pl.BlockSpec((1, dout), lambda i: (0, 0)),
        ],
        out_specs=pl.BlockSpec((block_rows, dout), lambda i: (i, 0)),
        out_shape=jax.ShapeDtypeStruct((m, dout), out_dtype),
    )(x, w, b.reshape(1, dout))


def _bf16_bits(y):
    i = lax.bitcast_convert_type(y, jnp.int32)
    return (i + jnp.int32(0x7FFF) + ((i >> 16) & 1)) >> 16


def _epack_body(x_ref, wa_ref, ba_ref, wb_ref, bb_ref, o_ref):
    x = x_ref[...]
    ya = jnp.dot(x, wa_ref[...], preferred_element_type=jnp.float32)
    ya = ya + ba_ref[...]
    yb = jnp.dot(x, wb_ref[...], preferred_element_type=jnp.float32)
    yb = yb + bb_ref[...]
    o_ref[...] = (_bf16_bits(ya) & jnp.int32(0xFFFF)) | (_bf16_bits(yb) << 16)


def _enc_body(ea_ref, wa_ref, ba_ref, wb_ref, bb_ref, x_ref, wx_ref, bx_ref,
              oe_ref, oh_ref, *, nh):
    _epack_body(ea_ref, wa_ref, ba_ref, wb_ref, bb_ref, oe_ref)

    @pl.when(pl.program_id(0) < nh)
    def _():
        y = jnp.dot(x_ref[...], wx_ref[...],
                    preferred_element_type=jnp.float32)
        oh_ref[...] = y + bx_ref[...]


def _encoders(edge_attr, wb, bb, x, wa, ba, eblock, hblock):
    m, k = edge_attr.shape
    grid = m // eblock
    nh = N // hblock
    hmap = lambda i: (jnp.minimum(i, nh - 1), 0)
    return pl.pallas_call(
        functools.partial(_enc_body, nh=nh),
        grid=(grid,),
        in_specs=[
            pl.BlockSpec((eblock, k), lambda i: (i, 0)),
            pl.BlockSpec((k, DP), lambda i: (0, 0)),
            pl.BlockSpec((1, DP), lambda i: (0, 0)),
            pl.BlockSpec((k, DP), lambda i: (0, 0)),
            pl.BlockSpec((1, DP), lambda i: (0, 0)),
            pl.BlockSpec((hblock, D), hmap),
            pl.BlockSpec((D, D), lambda i: (0, 0)),
            pl.BlockSpec((1, D), lambda i: (0, 0)),
        ],
        out_specs=[
            pl.BlockSpec((eblock, DP), lambda i: (i, 0)),
            pl.BlockSpec((hblock, D), hmap),
        ],
        out_shape=[
            jax.ShapeDtypeStruct((m, DP), jnp.int32),
            jax.ShapeDtypeStruct((N, D), jnp.float32),
        ],
    )(edge_attr, wb[:, PA], bb[PA].reshape(1, DP), wb[:, PB],
      bb[PB].reshape(1, DP), x, wa, ba.reshape(1, D))


def _mlp_body(h_ref, p_ref, w1_ref, b1_ref, g1_ref, be1_ref, w2_ref, b2_ref,
              o_ref, *, wl_ref=None, bl_ref=None):
    t = h_ref[...] + p_ref[0] + p_ref[1]
    t = jnp.dot(t, w1_ref[...], preferred_element_type=jnp.float32)
    t = t + b1_ref[...]
    mu = jnp.mean(t, axis=-1, keepdims=True)
    var = jnp.mean((t - mu) ** 2, axis=-1, keepdims=True)
    t = (t - mu) / jnp.sqrt(var + 1e-5) * g1_ref[...] + be1_ref[...]
    t = t * jax.nn.sigmoid(t)
    t = jnp.dot(t, w2_ref[...], preferred_element_type=jnp.float32)
    t = t + b2_ref[...]
    t = t * jax.nn.sigmoid(t)
    if wl_ref is not None:
        t = jnp.dot(t, wl_ref[...], preferred_element_type=jnp.float32)
        t = t + bl_ref[...]
        t = t * jax.nn.sigmoid(t)
    o_ref[...] = t


def _mlp_final_body(h_ref, p_ref, w1_ref, b1_ref, g1_ref, be1_ref, w2_ref,
                    b2_ref, wl_ref, bl_ref, o_ref):
    _mlp_body(h_ref, p_ref, w1_ref, b1_ref, g1_ref, be1_ref, w2_ref, b2_ref,
              o_ref, wl_ref=wl_ref, bl_ref=bl_ref)


def _mlp(h, parts, lp, final=None, block_rows=2000):
    vec = lambda v: v.reshape(1, D)
    mat_spec = pl.BlockSpec((D, D), lambda i: (0, 0))
    vec_spec = pl.BlockSpec((1, D), lambda i: (0, 0))
    row_spec = pl.BlockSpec((block_rows, D), lambda i: (i, 0))
    in_specs = [
        row_spec,
        pl.BlockSpec((2, block_rows, D), lambda i: (0, i, 0)),
        mat_spec, vec_spec, vec_spec, vec_spec, mat_spec, vec_spec,
    ]
    args = [h, parts, lp['W1'], vec(lp['b1']), vec(lp['g1']), vec(lp['be1']),
            lp['W2'], vec(lp['b2'])]
    body = _mlp_body
    if final is not None:
        wl, bl = final
        body = _mlp_final_body
        in_specs += [mat_spec, vec_spec]
        args += [wl, vec(bl)]
    return pl.pallas_call(
        body,
        grid=(N // block_rows,),
        in_specs=in_specs,
        out_specs=row_spec,
        out_shape=jax.ShapeDtypeStruct((N, D), jnp.float32),
    )(*args)


def _unpack_pair(u):
    a = lax.bitcast_convert_type(u << 16, jnp.float32)
    bb = lax.bitcast_convert_type(u & jnp.int32(-65536), jnp.float32)
    return a, bb


def _make_agg():
    mesh = plsc.VectorSubcoreMesh(core_axis_name="c", subcore_axis_name="s")

    @functools.partial(
        pl.kernel,
        mesh=mesh,
        out_type=jax.ShapeDtypeStruct((NC, N, D), jnp.float32),
        scratch_types=[
            pltpu.VMEM((B,), jnp.int32),
            pltpu.VMEM((B,), jnp.int32),
            pltpu.VMEM((B,), jnp.int32),
            pltpu.VMEM((B,), jnp.int32),
            pltpu.VMEM((B,), jnp.int32),
            pltpu.VMEM((B,), jnp.int32),
            pltpu.VMEM((B, D), jnp.float32),
            pltpu.VMEM((B, D), jnp.float32),
            pltpu.VMEM((B, DP), jnp.int32),
            pltpu.VMEM((B, DP), jnp.int32),
            pltpu.VMEM((REM,), jnp.int32),
            pltpu.VMEM((REM,), jnp.int32),
            pltpu.VMEM_SHARED((N, D), jnp.float32),
            pltpu.SemaphoreType.DMA,
            pltpu.SemaphoreType.DMA,
            pltpu.SemaphoreType.DMA,
            pltpu.SemaphoreType.DMA,
            pltpu.SemaphoreType.DMA,
            pltpu.SemaphoreType.DMA,
            pltpu.SemaphoreType.DMA,
            pltpu.SemaphoreType.DMA,
            pltpu.SemaphoreType.DMA,
        ],
    )
    def agg(h_hbm, e_hbm, src_hbm, dst_hbm, out_hbm,
            si0, si1, di0, di1, dsc0, dsc1, rows0, rows1, ev0, ev1,
            si2, di2, acc,
            ssi0, ssi1, sde0, sde1, sg0, sg1, ssc0, ssc1, sem2):
        SI = (si0, si1)
        DI = (di0, di1)
        DSC = (dsc0, dsc1)
        ROWS = (rows0, rows1)
        EV = (ev0, ev1)
        SSI = (ssi0, ssi1)
        SDE = (sde0, sde1)
        SG = (sg0, sg1)
        SSC = (ssc0, ssc1)

        cid = lax.axis_index("c")
        sid = lax.axis_index("s")
        wid = sid * NC + cid
        base = wid * EP

        def prefetch(k, b):
            off = pl.multiple_of(base + k * B, 8)
            pltpu.async_copy(src_hbm.at[pl.ds(off, B)], SI[b], SSI[b])
            pltpu.async_copy(dst_hbm.at[pl.ds(off, B)], DI[b], SDE[b])
            pltpu.async_copy(e_hbm.at[pl.ds(off, B)], EV[b], SDE[b])

        def wait_si(b):
            pltpu.make_async_copy(src_hbm.at[pl.ds(0, B)], SI[b], SSI[b]).wait()

        def wait_de(b):
            pltpu.make_async_copy(dst_hbm.at[pl.ds(0, B)], DI[b], SDE[b]).wait()
            pltpu.make_async_copy(e_hbm.at[pl.ds(0, B)], EV[b],
                                  SDE[b]).wait()

        def wait_gather(b):
            pltpu.make_async_copy(h_hbm.at[SI[b]], ROWS[b], SG[b]).wait()

        def wait_scatter(b):
            pltpu.make_async_copy(ROWS[b], acc.at[DSC[b]], SSC[b]).wait()

        prefetch(0, 0)
        prefetch(1, 1)

        zero16 = jnp.zeros((16,), jnp.float32)

        @plsc.parallel_loop(0, ZB)
        def _(i):
            for j in range(D // 16):
                rows0[i, pl.ds(j * 16, 16)] = zero16

        def zchunk(q, _):
            c = sid + q * NS

            @pl.when(c < NROWCH)
            def _():
                pltpu.sync_copy(rows0.at[pl.ds(0, ZB)],
                                acc.at[pl.ds(c * ZB, ZB)])
            return 0
        lax.fori_loop(0, (NROWCH + NS - 1) // NS, zchunk, 0)

        @pl.when(sid == 0)
        def _():
            pltpu.sync_copy(rows0.at[pl.ds(0, ROWREM)],
                            acc.at[pl.ds(NROWCH * ZB, ROWREM)])

        plsc.subcore_barrier()

        wait_si(0)
        pltpu.async_copy(h_hbm.at[SI[0]], ROWS[0], SG[0])

        def chunk_body(k, b, first, gather_guard, pref_guard):
            b1 = 1 - b

            def issue_next():
                wait_si(b1)
                if not first:
                    wait_scatter(b1)
                pltpu.async_copy(h_hbm.at[SI[b1]], ROWS[b1], SG[b1])

            if gather_guard is None:
                issue_next()
            else:
                @pl.when(gather_guard)
                def _():
                    issue_next()

            wait_de(b)
            wait_gather(b)

            @plsc.parallel_loop(0, B, unroll=4)
            def _(i):
                for jw in range(DP // 16):
                    ea, eb = _unpack_pair(EV[b][i, pl.ds(jw * 16, 16)])
                    sa = pl.ds(jw * 32, 16)
                    sb = pl.ds(jw * 32 + 16, 16)
                    ROWS[b][i, sa] = jnp.maximum(ROWS[b][i, sa] + ea, 0.0)
                    ROWS[b][i, sb] = jnp.maximum(ROWS[b][i, sb] + eb, 0.0)

            @plsc.parallel_loop(0, B, step=16)
            def _(i):
                DSC[b][pl.ds(i, 16)] = DI[b][pl.ds(i, 16)]

            pltpu.async_copy(ROWS[b], acc.at[DSC[b]], SSC[b], add=True)

            def issue_pref():
                prefetch(k + 2, b)

            if pref_guard is None:
                issue_pref()
            else:
                @pl.when(pref_guard)
                def _():
                    issue_pref()

        chunk_body(0, 0, True, None, None)
        chunk_body(1, 1, False, None, None)

        NPAIR = NFULL // 2

        def pair(q, _):
            k0 = q * 2
            tail = q < NPAIR - 1
            chunk_body(k0, 0, False, None, tail)
            chunk_body(k0 + 1, 1, False, tail, tail)
            return 0
        lax.fori_loop(1, NPAIR, pair, 0)

        wait_scatter(0)
        wait_scatter(1)

        off = base + NFULL * B
        pltpu.sync_copy(src_hbm.at[pl.ds(off, REM)], si2)
        gather = pltpu.async_copy(h_hbm.at[si2], rows0.at[pl.ds(0, REM)],
                                  sem2)
        pltpu.sync_copy(dst_hbm.at[pl.ds(off, REM)], di2)
        pltpu.sync_copy(e_hbm.at[pl.ds(off, REM)], ev0.at[pl.ds(0, REM)])
        gather.wait()

        @plsc.parallel_loop(0, REM)
        def _(i):
            for jw in range(DP // 16):
                ea, eb = _unpack_pair(ev0[i, pl.ds(jw * 16, 16)])
                sa = pl.ds(jw * 32, 16)
                sb = pl.ds(jw * 32 + 16, 16)
                rows0[i, sa] = jnp.maximum(rows0[i, sa] + ea, 0.0)
                rows0[i, sb] = jnp.maximum(rows0[i, sb] + eb, 0.0)

        pltpu.sync_copy(rows0.at[pl.ds(0, REM)], acc.at[di2], add=True)

        plsc.subcore_barrier()

        def ochunk(q, _):
            c = sid + q * NS

            @pl.when(c < NROWCH)
            def _():
                pltpu.sync_copy(acc.at[pl.ds(c * ZB, ZB)],
                                out_hbm.at[cid, pl.ds(c * ZB, ZB)])
            return 0
        lax.fori_loop(0, (NROWCH + NS - 1) // NS, ochunk, 0)

        @pl.when(sid == 0)
        def _():
            pltpu.sync_copy(acc.at[pl.ds(NROWCH * ZB, ROWREM)],
                            out_hbm.at[cid, pl.ds(NROWCH * ZB, ROWREM)])

    return agg


def kernel(x, edge_index, edge_attr, batch, params):
    p = params
    src = edge_index[0]
    dst = edge_index[1]

    ep, h = _encoders(edge_attr, p['Wb'], p['bb'], x, p['Wa'], p['ba'],
                      eblock=4000, hblock=2000)

    agg = _make_agg()
    nl = len(p['layers'])
    for i, lp in enumerate(p['layers']):
        parts = agg(h, ep, src, dst)
        final = (p['Wl'], p['bl']) if i + 1 == nl else None
        h = _mlp(h, parts, lp, final=final)

    return h
